# baseline (speedup 1.0000x reference)
import jax
import jax.numpy as jnp
import numpy as np
from jax import lax
from jax.experimental import pallas as pl
from jax.experimental.pallas import tpu as pltpu

K = 9
DILATION = 2
KK = K * DILATION - 1
TILE = 512
OUTW = 16
LANES = 128
DEPTH = 4
INF = np.float32(np.inf)


def _knn_kernel(xt_ref, xrow_ref, out_ref, dist_ref):
    xall = xt_ref[0]
    n = xall.shape[0]
    nchunks = n // LANES
    norm = jnp.sqrt(jnp.sum(xall * xall, axis=1, keepdims=True))
    xn = xall / jnp.maximum(norm, 1e-12)
    xr = xrow_ref[0]
    rnorm = jnp.sqrt(jnp.sum(xr * xr, axis=1, keepdims=True))
    rows = xr / jnp.maximum(rnorm, 1e-12)
    inner = -2.0 * lax.dot_general(
        rows, xn, (((1,), (1,)), ((), ())),
        preferred_element_type=jnp.float32)
    sq_rows = jnp.sum(rows * rows, axis=1, keepdims=True)
    sq_all = jnp.sum(xn * xn, axis=1)[None, :]
    dist_ref[...] = (sq_rows + inner) + sq_all

    lane_f = lax.broadcasted_iota(
        jnp.int32, (TILE, LANES), 1).astype(jnp.float32)
    biga = jnp.float32(2 * n)

    def scan_chunk(c, carry):
        m, a = carry
        x = dist_ref[:, pl.ds(c * LANES, LANES)]
        fx = lane_f + (c * LANES).astype(jnp.float32)
        lt = [x < m[i] for i in range(DEPTH)]
        nm = [jnp.where(lt[0], x, m[0])]
        na = [jnp.where(lt[0], fx, a[0])]
        for i in range(1, DEPTH):
            nm.append(jnp.where(lt[i - 1], m[i - 1],
                                jnp.where(lt[i], x, m[i])))
            na.append(jnp.where(lt[i - 1], a[i - 1],
                                jnp.where(lt[i], fx, a[i])))
        return tuple(nm), tuple(na)

    m0 = tuple(jnp.full((TILE, LANES), INF) for _ in range(DEPTH))
    a0 = tuple(jnp.full((TILE, LANES), biga) for _ in range(DEPTH))
    m, a = lax.fori_loop(0, nchunks, scan_chunk, (m0, a0))
    m, a = list(m), list(a)

    cols = []
    for t in range(KK):
        v = jnp.min(m[0], axis=1, keepdims=True)
        fidx = jnp.min(jnp.where(m[0] == v, a[0], biga), axis=1)
        if t % DILATION == 0:
            cols.append(fidx)
        if t < KK - 1:
            hit = a[0] == fidx[:, None]
            for i in range(DEPTH - 1):
                m[i] = jnp.where(hit, m[i + 1], m[i])
                a[i] = jnp.where(hit, a[i + 1], a[i])
            m[DEPTH - 1] = jnp.where(hit, INF, m[DEPTH - 1])
            a[DEPTH - 1] = jnp.where(hit, biga, a[DEPTH - 1])
    fast = jnp.stack(cols, axis=1)

    suspect = jnp.any(m[0] == INF, axis=1) | (fast[:, K - 1] >= biga)
    out_ref[0] = jnp.pad(fast.astype(jnp.int32), ((0, 0), (0, OUTW - K)))

    def _fallback():
        fiota = lax.broadcasted_iota(
            jnp.int32, (TILE, n), 1).astype(jnp.float32)
        big = jnp.float32(n)
        slow_cols = []
        for t in range(KK):
            d = dist_ref[...]
            v = jnp.min(d, axis=1, keepdims=True)
            fidx = jnp.min(jnp.where(d == v, fiota, big), axis=1)
            if t % DILATION == 0:
                slow_cols.append(fidx)
            if t < KK - 1:
                dist_ref[...] = jnp.where(fiota == fidx[:, None], INF, d)
        slow = jnp.stack(slow_cols, axis=1)
        fixed = jnp.where(suspect[:, None], slow, fast)
        out_ref[0] = jnp.pad(fixed.astype(jnp.int32),
                             ((0, 0), (0, OUTW - K)))

    pl.when(jnp.any(suspect))(_fallback)


def kernel(x):
    b, c, n, _ = x.shape
    xt = jnp.transpose(x[..., 0], (0, 2, 1))
    nn = pl.pallas_call(
        _knn_kernel,
        grid=(b, n // TILE),
        in_specs=[pl.BlockSpec((1, n, c), lambda bb, ii: (bb, 0, 0)),
                  pl.BlockSpec((1, TILE, c), lambda bb, ii: (bb, ii, 0))],
        out_specs=pl.BlockSpec((1, TILE, OUTW), lambda bb, ii: (bb, ii, 0)),
        out_shape=jax.ShapeDtypeStruct((b, n, OUTW), jnp.int32),
        scratch_shapes=[pltpu.VMEM((TILE, n), jnp.float32)],
    )(xt, xt)
    nn9 = nn[..., :K]
    center = jnp.broadcast_to(
        jnp.arange(n, dtype=jnp.int32)[None, :, None], (b, n, K))
    return jnp.stack((nn9, center), axis=0)


# speedup vs baseline: 1.0611x; 1.0611x over previous
import jax
import jax.numpy as jnp
import numpy as np
from jax import lax
from jax.experimental import pallas as pl
from jax.experimental.pallas import tpu as pltpu

K = 9
DILATION = 2
KK = K * DILATION - 1
TILE = 512
OUTW = 16
LANES = 128
DEPTH = 4
INF = np.float32(np.inf)


def _knn_kernel(xt_ref, xrow_ref, out_ref, dist_ref):
    xall = xt_ref[0]
    n = xall.shape[0]
    nchunks = n // LANES
    norm = jnp.sqrt(jnp.sum(xall * xall, axis=1, keepdims=True))
    xn = xall / jnp.maximum(norm, 1e-12)
    xr = xrow_ref[0]
    rnorm = jnp.sqrt(jnp.sum(xr * xr, axis=1, keepdims=True))
    rows = xr / jnp.maximum(rnorm, 1e-12)
    inner = -2.0 * lax.dot_general(
        rows, xn, (((1,), (1,)), ((), ())),
        preferred_element_type=jnp.float32)
    sq_rows = jnp.sum(rows * rows, axis=1, keepdims=True)
    sq_all = jnp.sum(xn * xn, axis=1)[None, :]
    dist_ref[...] = (sq_rows + inner) + sq_all

    lane_f = lax.broadcasted_iota(
        jnp.int32, (TILE, LANES), 1).astype(jnp.float32)
    biga = jnp.float32(2 * n)

    m = [jnp.full((TILE, LANES), INF) for _ in range(DEPTH)]
    a = [jnp.full((TILE, LANES), biga) for _ in range(DEPTH)]
    for c in range(nchunks):
        x = dist_ref[:, c * LANES:(c + 1) * LANES]
        fx = lane_f + jnp.float32(c * LANES)
        lt = [x < m[i] for i in range(DEPTH)]
        nm = [jnp.where(lt[0], x, m[0])]
        na = [jnp.where(lt[0], fx, a[0])]
        for i in range(1, DEPTH):
            nm.append(jnp.where(lt[i - 1], m[i - 1],
                                jnp.where(lt[i], x, m[i])))
            na.append(jnp.where(lt[i - 1], a[i - 1],
                                jnp.where(lt[i], fx, a[i])))
        m, a = nm, na

    cols = []
    for t in range(KK):
        v = jnp.min(m[0], axis=1, keepdims=True)
        fidx = jnp.min(jnp.where(m[0] == v, a[0], biga), axis=1)
        if t % DILATION == 0:
            cols.append(fidx)
        if t < KK - 1:
            hit = a[0] == fidx[:, None]
            for i in range(DEPTH - 1):
                m[i] = jnp.where(hit, m[i + 1], m[i])
                a[i] = jnp.where(hit, a[i + 1], a[i])
            m[DEPTH - 1] = jnp.where(hit, INF, m[DEPTH - 1])
            a[DEPTH - 1] = jnp.where(hit, biga, a[DEPTH - 1])
    fast = jnp.stack(cols, axis=1)

    suspect = jnp.any(m[0] == INF, axis=1) | (fast[:, K - 1] >= biga)
    out_ref[0] = jnp.pad(fast.astype(jnp.int32), ((0, 0), (0, OUTW - K)))

    def _fallback():
        fiota = lax.broadcasted_iota(
            jnp.int32, (TILE, n), 1).astype(jnp.float32)
        big = jnp.float32(n)
        slow_cols = []
        for t in range(KK):
            d = dist_ref[...]
            v = jnp.min(d, axis=1, keepdims=True)
            fidx = jnp.min(jnp.where(d == v, fiota, big), axis=1)
            if t % DILATION == 0:
                slow_cols.append(fidx)
            if t < KK - 1:
                dist_ref[...] = jnp.where(fiota == fidx[:, None], INF, d)
        slow = jnp.stack(slow_cols, axis=1)
        fixed = jnp.where(suspect[:, None], slow, fast)
        out_ref[0] = jnp.pad(fixed.astype(jnp.int32),
                             ((0, 0), (0, OUTW - K)))

    pl.when(jnp.any(suspect))(_fallback)


def kernel(x):
    b, c, n, _ = x.shape
    xt = jnp.transpose(x[..., 0], (0, 2, 1))
    nn = pl.pallas_call(
        _knn_kernel,
        grid=(b, n // TILE),
        in_specs=[pl.BlockSpec((1, n, c), lambda bb, ii: (bb, 0, 0)),
                  pl.BlockSpec((1, TILE, c), lambda bb, ii: (bb, ii, 0))],
        out_specs=pl.BlockSpec((1, TILE, OUTW), lambda bb, ii: (bb, ii, 0)),
        out_shape=jax.ShapeDtypeStruct((b, n, OUTW), jnp.int32),
        scratch_shapes=[pltpu.VMEM((TILE, n), jnp.float32)],
    )(xt, xt)
    nn9 = nn[..., :K]
    center = jnp.broadcast_to(
        jnp.arange(n, dtype=jnp.int32)[None, :, None], (b, n, K))
    return jnp.stack((nn9, center), axis=0)


# speedup vs baseline: 1.0614x; 1.0002x over previous
import jax
import jax.numpy as jnp
import numpy as np
from jax import lax
from jax.experimental import pallas as pl
from jax.experimental.pallas import tpu as pltpu

K = 9
DILATION = 2
KK = K * DILATION - 1
TILE = 64
OUTW = 16
LANES = 128
DEPTH = 4
INF = np.float32(np.inf)


def _knn_kernel(xt_ref, xrow_ref, out_ref, dist_ref):
    xall = xt_ref[0]
    n = xall.shape[0]
    nchunks = n // LANES
    norm = jnp.sqrt(jnp.sum(xall * xall, axis=1, keepdims=True))
    xn = xall / jnp.maximum(norm, 1e-12)
    xr = xrow_ref[0]
    rnorm = jnp.sqrt(jnp.sum(xr * xr, axis=1, keepdims=True))
    rows = xr / jnp.maximum(rnorm, 1e-12)
    inner = -2.0 * lax.dot_general(
        rows, xn, (((1,), (1,)), ((), ())),
        preferred_element_type=jnp.float32)
    sq_rows = jnp.sum(rows * rows, axis=1, keepdims=True)
    sq_all = jnp.sum(xn * xn, axis=1)[None, :]
    dist_ref[...] = (sq_rows + inner) + sq_all

    lane_f = lax.broadcasted_iota(
        jnp.int32, (TILE, LANES), 1).astype(jnp.float32)
    biga = jnp.float32(2 * n)

    m = [jnp.full((TILE, LANES), INF) for _ in range(DEPTH)]
    a = [jnp.full((TILE, LANES), biga) for _ in range(DEPTH)]
    for c in range(nchunks):
        x = dist_ref[:, c * LANES:(c + 1) * LANES]
        fx = lane_f + jnp.float32(c * LANES)
        lt = [x < m[i] for i in range(DEPTH)]
        nm = [jnp.where(lt[0], x, m[0])]
        na = [jnp.where(lt[0], fx, a[0])]
        for i in range(1, DEPTH):
            nm.append(jnp.where(lt[i - 1], m[i - 1],
                                jnp.where(lt[i], x, m[i])))
            na.append(jnp.where(lt[i - 1], a[i - 1],
                                jnp.where(lt[i], fx, a[i])))
        m, a = nm, na

    cols = []
    for t in range(KK):
        v = jnp.min(m[0], axis=1, keepdims=True)
        fidx = jnp.min(jnp.where(m[0] == v, a[0], biga), axis=1)
        if t % DILATION == 0:
            cols.append(fidx)
        if t < KK - 1:
            hit = a[0] == fidx[:, None]
            for i in range(DEPTH - 1):
                m[i] = jnp.where(hit, m[i + 1], m[i])
                a[i] = jnp.where(hit, a[i + 1], a[i])
            m[DEPTH - 1] = jnp.where(hit, INF, m[DEPTH - 1])
            a[DEPTH - 1] = jnp.where(hit, biga, a[DEPTH - 1])
    fast = jnp.stack(cols, axis=1)

    suspect = jnp.any(m[0] == INF, axis=1) | (fast[:, K - 1] >= biga)
    out_ref[0] = jnp.pad(fast.astype(jnp.int32), ((0, 0), (0, OUTW - K)))

    def _fallback():
        fiota = lax.broadcasted_iota(
            jnp.int32, (TILE, n), 1).astype(jnp.float32)
        big = jnp.float32(n)
        slow_cols = []
        for t in range(KK):
            d = dist_ref[...]
            v = jnp.min(d, axis=1, keepdims=True)
            fidx = jnp.min(jnp.where(d == v, fiota, big), axis=1)
            if t % DILATION == 0:
                slow_cols.append(fidx)
            if t < KK - 1:
                dist_ref[...] = jnp.where(fiota == fidx[:, None], INF, d)
        slow = jnp.stack(slow_cols, axis=1)
        fixed = jnp.where(suspect[:, None], slow, fast)
        out_ref[0] = jnp.pad(fixed.astype(jnp.int32),
                             ((0, 0), (0, OUTW - K)))

    pl.when(jnp.any(suspect))(_fallback)


def kernel(x):
    b, c, n, _ = x.shape
    xt = jnp.transpose(x[..., 0], (0, 2, 1))
    nn = pl.pallas_call(
        _knn_kernel,
        grid=(b, n // TILE),
        in_specs=[pl.BlockSpec((1, n, c), lambda bb, ii: (bb, 0, 0)),
                  pl.BlockSpec((1, TILE, c), lambda bb, ii: (bb, ii, 0))],
        out_specs=pl.BlockSpec((1, TILE, OUTW), lambda bb, ii: (bb, ii, 0)),
        out_shape=jax.ShapeDtypeStruct((b, n, OUTW), jnp.int32),
        scratch_shapes=[pltpu.VMEM((TILE, n), jnp.float32)],
    )(xt, xt)
    nn9 = nn[..., :K]
    center = jnp.broadcast_to(
        jnp.arange(n, dtype=jnp.int32)[None, :, None], (b, n, K))
    return jnp.stack((nn9, center), axis=0)


# speedup vs baseline: 1.3685x; 1.2893x over previous
import jax
import jax.numpy as jnp
import numpy as np
from jax import lax
from jax.experimental import pallas as pl

K = 9
DILATION = 2
KK = K * DILATION - 1
TILE = 64
OUTW = 16
LANES = 128
DEPTH = 4
INF = np.float32(np.inf)


def _normalize_kernel(xt_ref, xn_ref, sq_ref):
    x = xt_ref[0]
    norm = jnp.sqrt(jnp.sum(x * x, axis=1, keepdims=True))
    xn = x / jnp.maximum(norm, 1e-12)
    xn_ref[0] = xn
    sq = jnp.sum(xn * xn, axis=1)
    sq_ref[0] = jnp.broadcast_to(sq[None, :], (8, xn.shape[0]))


def _dist(rows, xn, sq_rows, sq_all):
    inner = -2.0 * lax.dot_general(
        rows, xn, (((1,), (1,)), ((), ())),
        preferred_element_type=jnp.float32)
    return (sq_rows + inner) + sq_all


def _knn_kernel(xn_ref, xrow_ref, sq_ref, out_ref):
    xn = xn_ref[0]
    n = xn.shape[0]
    nchunks = n // LANES
    rows = xrow_ref[0]
    sq_all = sq_ref[0][0][None, :]
    sq_rows = jnp.sum(rows * rows, axis=1, keepdims=True)
    dist = _dist(rows, xn, sq_rows, sq_all)

    lane_f = lax.broadcasted_iota(
        jnp.int32, (TILE, LANES), 1).astype(jnp.float32)
    biga = jnp.float32(2 * n)

    m = [jnp.full((TILE, LANES), INF) for _ in range(DEPTH)]
    a = [jnp.full((TILE, LANES), biga) for _ in range(DEPTH)]
    for c in range(nchunks):
        x = dist[:, c * LANES:(c + 1) * LANES]
        fx = lane_f + jnp.float32(c * LANES)
        lt = [x < m[i] for i in range(DEPTH)]
        nm = [jnp.where(lt[0], x, m[0])]
        na = [jnp.where(lt[0], fx, a[0])]
        for i in range(1, DEPTH):
            nm.append(jnp.where(lt[i - 1], m[i - 1],
                                jnp.where(lt[i], x, m[i])))
            na.append(jnp.where(lt[i - 1], a[i - 1],
                                jnp.where(lt[i], fx, a[i])))
        m, a = nm, na

    cols = []
    for t in range(KK):
        v = jnp.min(m[0], axis=1, keepdims=True)
        fidx = jnp.min(jnp.where(m[0] == v, a[0], biga), axis=1)
        if t % DILATION == 0:
            cols.append(fidx)
        if t < KK - 1:
            hit = a[0] == fidx[:, None]
            for i in range(DEPTH - 1):
                m[i] = jnp.where(hit, m[i + 1], m[i])
                a[i] = jnp.where(hit, a[i + 1], a[i])
            m[DEPTH - 1] = jnp.where(hit, INF, m[DEPTH - 1])
            a[DEPTH - 1] = jnp.where(hit, biga, a[DEPTH - 1])
    fast = jnp.stack(cols, axis=1)

    suspect = jnp.any(m[0] == INF, axis=1) | (fast[:, K - 1] >= biga)
    out_ref[0] = jnp.pad(fast.astype(jnp.int32), ((0, 0), (0, OUTW - K)))

    def _fallback():
        d = _dist(rows, xn, sq_rows, sq_all)
        fiota = lax.broadcasted_iota(
            jnp.int32, (TILE, n), 1).astype(jnp.float32)
        big = jnp.float32(n)
        slow_cols = []
        for t in range(KK):
            v = jnp.min(d, axis=1, keepdims=True)
            fidx = jnp.min(jnp.where(d == v, fiota, big), axis=1)
            if t % DILATION == 0:
                slow_cols.append(fidx)
            if t < KK - 1:
                d = jnp.where(fiota == fidx[:, None], INF, d)
        slow = jnp.stack(slow_cols, axis=1)
        fixed = jnp.where(suspect[:, None], slow, fast)
        out_ref[0] = jnp.pad(fixed.astype(jnp.int32),
                             ((0, 0), (0, OUTW - K)))

    pl.when(jnp.any(suspect))(_fallback)


def kernel(x):
    b, c, n, _ = x.shape
    xt = jnp.transpose(x[..., 0], (0, 2, 1))
    xn, sq = pl.pallas_call(
        _normalize_kernel,
        grid=(b,),
        in_specs=[pl.BlockSpec((1, n, c), lambda bb: (bb, 0, 0))],
        out_specs=[pl.BlockSpec((1, n, c), lambda bb: (bb, 0, 0)),
                   pl.BlockSpec((1, 8, n), lambda bb: (bb, 0, 0))],
        out_shape=[jax.ShapeDtypeStruct((b, n, c), jnp.float32),
                   jax.ShapeDtypeStruct((b, 8, n), jnp.float32)],
    )(xt)
    nn = pl.pallas_call(
        _knn_kernel,
        grid=(b, n // TILE),
        in_specs=[pl.BlockSpec((1, n, c), lambda bb, ii: (bb, 0, 0)),
                  pl.BlockSpec((1, TILE, c), lambda bb, ii: (bb, ii, 0)),
                  pl.BlockSpec((1, 8, n), lambda bb, ii: (bb, 0, 0))],
        out_specs=pl.BlockSpec((1, TILE, OUTW), lambda bb, ii: (bb, ii, 0)),
        out_shape=jax.ShapeDtypeStruct((b, n, OUTW), jnp.int32),
    )(xn, xn, sq)
    nn9 = nn[..., :K]
    center = jnp.broadcast_to(
        jnp.arange(n, dtype=jnp.int32)[None, :, None], (b, n, K))
    return jnp.stack((nn9, center), axis=0)
